# Initial kernel scaffold; baseline (speedup 1.0000x reference)
import jax
import jax.numpy as jnp
from jax.experimental import pallas as pl


def kernel(x, eidx, enorm):
    raise NotImplementedError("write your pallas kernel here")


# speedup vs baseline: 2.4597x; 2.4597x over previous
import functools

import jax
import jax.numpy as jnp
from jax import lax
from jax.experimental import pallas as pl
from jax.experimental.pallas import tpu as pltpu
from jax.experimental.pallas import tpu_sc as plsc

_C = 128
_NS = 16
_LANES = 16


def _gc_body(NCH, RP, JB, x0, x1, si, ti, en, out0, out1,
             si_v, ti_v, en_v, rows_v, acc, sem):
    c = lax.axis_index("c")
    s = lax.axis_index("s")

    def _zrow(i, carry):
        for j in range(JB):
            rows_v[i, pl.ds(j * _LANES, _LANES)] = jnp.zeros((_LANES,), jnp.float32)
        return carry
    lax.fori_loop(0, _C, _zrow, 0)
    row0 = pl.multiple_of(s * RP, 8)
    full, rem = divmod(RP, _C)
    for kblk in range(full):
        pltpu.sync_copy(rows_v, acc.at[pl.ds(row0 + kblk * _C, _C)])
    if rem:
        pltpu.sync_copy(rows_v.at[pl.ds(0, rem)],
                        acc.at[pl.ds(row0 + full * _C, rem)])
    plsc.subcore_barrier()

    EPT = NCH * _C

    def _chunk(g, carry):
        base = pl.multiple_of(s * EPT + g * _C, _C)
        pltpu.sync_copy(si.at[pl.ds(base, _C)], si_v)
        pltpu.sync_copy(ti.at[pl.ds(base, _C)], ti_v)
        enrow = pl.multiple_of(base // 8, _C // 8)
        pltpu.sync_copy(en.at[pl.ds(enrow, _C // 8)], en_v)

        @pl.when(c == 0)
        def _():
            pltpu.async_copy(x0.at[si_v], rows_v, sem).wait()

        @pl.when(c == 1)
        def _():
            pltpu.async_copy(x1.at[si_v], rows_v, sem).wait()

        def _scale(e, carry2):
            sc16 = en_v[e // 8, pl.ds((e % 8) * _LANES, _LANES)]
            for j in range(JB):
                sl = pl.ds(j * _LANES, _LANES)
                rows_v[e, sl] = rows_v[e, sl] * sc16
            return carry2
        lax.fori_loop(0, _C, _scale, 0)

        pltpu.sync_copy(rows_v, acc.at[ti_v], add=True)
        return carry
    lax.fori_loop(0, NCH, _chunk, 0)

    plsc.subcore_barrier()

    @pl.when(c == 0)
    def _():
        pltpu.sync_copy(acc.at[pl.ds(row0, RP)], out0.at[pl.ds(row0, RP)])

    @pl.when(c == 1)
    def _():
        pltpu.sync_copy(acc.at[pl.ds(row0, RP)], out1.at[pl.ds(row0, RP)])


def kernel(x, eidx, enorm):
    N, D = x.shape
    E = eidx.shape[1]
    Dh = D // 2
    EPAD = -(-E // (_NS * _C)) * (_NS * _C)
    NCH = EPAD // (_NS * _C)
    NP = -(-N // (_NS * 8)) * (_NS * 8)
    RP = NP // _NS

    si = jnp.pad(eidx[0].astype(jnp.int32), (0, EPAD - E))
    ti = jnp.pad(eidx[1].astype(jnp.int32), (0, EPAD - E))
    en = jnp.broadcast_to(
        jnp.pad(enorm, (0, EPAD - E))[:, None], (EPAD, _LANES)
    ).reshape(EPAD // 8, 8 * _LANES)
    x0 = x[:, :Dh]
    x1 = x[:, Dh:]

    mesh = plsc.VectorSubcoreMesh(core_axis_name="c", subcore_axis_name="s")
    out0, out1 = pl.kernel(
        functools.partial(_gc_body, NCH, RP, Dh // _LANES),
        out_type=(jax.ShapeDtypeStruct((NP, Dh), jnp.float32),
                  jax.ShapeDtypeStruct((NP, Dh), jnp.float32)),
        mesh=mesh,
        scratch_types=[
            pltpu.VMEM((_C,), jnp.int32),
            pltpu.VMEM((_C,), jnp.int32),
            pltpu.VMEM((_C // 8, 8 * _LANES), jnp.float32),
            pltpu.VMEM((_C, Dh), jnp.float32),
            pltpu.VMEM_SHARED((NP, Dh), jnp.float32),
            pltpu.SemaphoreType.DMA,
        ],
    )(x0, x1, si, ti, en)
    return jnp.concatenate([out0[:N], out1[:N]], axis=1)


# speedup vs baseline: 2.6550x; 1.0794x over previous
import functools

import jax
import jax.numpy as jnp
from jax import lax
from jax.experimental import pallas as pl
from jax.experimental.pallas import tpu as pltpu
from jax.experimental.pallas import tpu_sc as plsc

_C = 128
_NS = 16
_LANES = 16


def _gc_body(NCH, RP, JB, x0, x1, si, ti, en, out0, out1,
             si_v, ti_v, en_v, rows_v, acc, sem):
    c = lax.axis_index("c")
    s = lax.axis_index("s")

    def _zrow(i, carry):
        for j in range(JB):
            rows_v[i, pl.ds(j * _LANES, _LANES)] = jnp.zeros((_LANES,), jnp.float32)
        return carry
    lax.fori_loop(0, _C, _zrow, 0)
    row0 = pl.multiple_of(s * RP, 8)
    full, rem = divmod(RP, _C)
    for kblk in range(full):
        pltpu.sync_copy(rows_v, acc.at[pl.ds(row0 + kblk * _C, _C)])
    if rem:
        pltpu.sync_copy(rows_v.at[pl.ds(0, rem)],
                        acc.at[pl.ds(row0 + full * _C, rem)])
    plsc.subcore_barrier()

    EPT = NCH * _C

    def _chunk(g, carry):
        base = pl.multiple_of(s * EPT + g * _C, _C)
        pltpu.sync_copy(si.at[pl.ds(base, _C)], si_v)
        pltpu.sync_copy(ti.at[pl.ds(base, _C)], ti_v)
        enrow = pl.multiple_of(base // 8, _C // 8)
        pltpu.sync_copy(en.at[pl.ds(enrow, _C // 8)], en_v)

        @pl.when(c == 0)
        def _():
            pltpu.async_copy(x0.at[si_v], rows_v, sem).wait()

        @pl.when(c == 1)
        def _():
            pltpu.async_copy(x1.at[si_v], rows_v, sem).wait()

        @plsc.parallel_loop(0, _C, unroll=8)
        def _scale(e):
            sc16 = en_v[e // 8, pl.ds((e % 8) * _LANES, _LANES)]
            for j in range(JB):
                sl = pl.ds(j * _LANES, _LANES)
                rows_v[e, sl] = rows_v[e, sl] * sc16

        pltpu.sync_copy(rows_v, acc.at[ti_v], add=True)
        return carry
    lax.fori_loop(0, NCH, _chunk, 0)

    plsc.subcore_barrier()

    @pl.when(c == 0)
    def _():
        pltpu.sync_copy(acc.at[pl.ds(row0, RP)], out0.at[pl.ds(row0, RP)])

    @pl.when(c == 1)
    def _():
        pltpu.sync_copy(acc.at[pl.ds(row0, RP)], out1.at[pl.ds(row0, RP)])


def kernel(x, eidx, enorm):
    N, D = x.shape
    E = eidx.shape[1]
    Dh = D // 2
    EPAD = -(-E // (_NS * _C)) * (_NS * _C)
    NCH = EPAD // (_NS * _C)
    NP = -(-N // (_NS * 8)) * (_NS * 8)
    RP = NP // _NS

    si = jnp.pad(eidx[0].astype(jnp.int32), (0, EPAD - E))
    ti = jnp.pad(eidx[1].astype(jnp.int32), (0, EPAD - E))
    en = jnp.broadcast_to(
        jnp.pad(enorm, (0, EPAD - E))[:, None], (EPAD, _LANES)
    ).reshape(EPAD // 8, 8 * _LANES)
    x0 = x[:, :Dh]
    x1 = x[:, Dh:]

    mesh = plsc.VectorSubcoreMesh(core_axis_name="c", subcore_axis_name="s")
    out0, out1 = pl.kernel(
        functools.partial(_gc_body, NCH, RP, Dh // _LANES),
        out_type=(jax.ShapeDtypeStruct((NP, Dh), jnp.float32),
                  jax.ShapeDtypeStruct((NP, Dh), jnp.float32)),
        mesh=mesh,
        scratch_types=[
            pltpu.VMEM((_C,), jnp.int32),
            pltpu.VMEM((_C,), jnp.int32),
            pltpu.VMEM((_C // 8, 8 * _LANES), jnp.float32),
            pltpu.VMEM((_C, Dh), jnp.float32),
            pltpu.VMEM_SHARED((NP, Dh), jnp.float32),
            pltpu.SemaphoreType.DMA,
        ],
    )(x0, x1, si, ti, en)
    return jnp.concatenate([out0[:N], out1[:N]], axis=1)


# speedup vs baseline: 3.0958x; 1.1660x over previous
import functools

import jax
import jax.numpy as jnp
from jax import lax
from jax.experimental import pallas as pl
from jax.experimental.pallas import tpu as pltpu
from jax.experimental.pallas import tpu_sc as plsc

_C = 128
_NS = 16
_LANES = 16


def _gc_body(NCH, RP, JB, x0, x1, si, ti, en, out0, out1,
             si_v, ti_v, en_v, rows_v, acc, sem):
    c = lax.axis_index("c")
    s = lax.axis_index("s")

    def _zrow(i, carry):
        for j in range(JB):
            rows_v[i, pl.ds(j * _LANES, _LANES)] = jnp.zeros((_LANES,), jnp.float32)
        return carry
    lax.fori_loop(0, _C, _zrow, 0)
    row0 = pl.multiple_of(s * RP, 8)
    full, rem = divmod(RP, _C)
    for kblk in range(full):
        pltpu.sync_copy(rows_v, acc.at[pl.ds(row0 + kblk * _C, _C)])
    if rem:
        pltpu.sync_copy(rows_v.at[pl.ds(0, rem)],
                        acc.at[pl.ds(row0 + full * _C, rem)])
    plsc.subcore_barrier()

    EPT = NCH * _C

    def _chunk(g, carry):
        base = pl.multiple_of(s * EPT + g * _C, _C)
        pltpu.sync_copy(si.at[pl.ds(base, _C)], si_v)
        pltpu.sync_copy(ti.at[pl.ds(base, _C)], ti_v)
        pltpu.sync_copy(en.at[pl.ds(base, _C)], en_v)

        @pl.when(c == 0)
        def _():
            pltpu.async_copy(x0.at[si_v], rows_v, sem).wait()

        @pl.when(c == 1)
        def _():
            pltpu.async_copy(x1.at[si_v], rows_v, sem).wait()

        @plsc.parallel_loop(0, _C // _LANES, unroll=2)
        def _scale(k):
            en16 = en_v[pl.ds(k * _LANES, _LANES)]
            for l in range(_LANES):
                sc16 = jnp.take_along_axis(
                    en16, jnp.full((_LANES,), l, jnp.int32), axis=0,
                    mode="promise_in_bounds")
                e = k * _LANES + l
                for j in range(JB):
                    sl = pl.ds(j * _LANES, _LANES)
                    rows_v[e, sl] = rows_v[e, sl] * sc16

        pltpu.sync_copy(rows_v, acc.at[ti_v], add=True)
        return carry
    lax.fori_loop(0, NCH, _chunk, 0)

    plsc.subcore_barrier()

    @pl.when(c == 0)
    def _():
        pltpu.sync_copy(acc.at[pl.ds(row0, RP)], out0.at[pl.ds(row0, RP)])

    @pl.when(c == 1)
    def _():
        pltpu.sync_copy(acc.at[pl.ds(row0, RP)], out1.at[pl.ds(row0, RP)])


def kernel(x, eidx, enorm):
    N, D = x.shape
    E = eidx.shape[1]
    Dh = D // 2
    EPAD = -(-E // (_NS * _C)) * (_NS * _C)
    NCH = EPAD // (_NS * _C)
    NP = -(-N // (_NS * 8)) * (_NS * 8)
    RP = NP // _NS

    si = jnp.pad(eidx[0].astype(jnp.int32), (0, EPAD - E))
    ti = jnp.pad(eidx[1].astype(jnp.int32), (0, EPAD - E))
    en = jnp.pad(enorm, (0, EPAD - E))
    x0 = x[:, :Dh]
    x1 = x[:, Dh:]

    mesh = plsc.VectorSubcoreMesh(core_axis_name="c", subcore_axis_name="s")
    out0, out1 = pl.kernel(
        functools.partial(_gc_body, NCH, RP, Dh // _LANES),
        out_type=(jax.ShapeDtypeStruct((NP, Dh), jnp.float32),
                  jax.ShapeDtypeStruct((NP, Dh), jnp.float32)),
        mesh=mesh,
        scratch_types=[
            pltpu.VMEM((_C,), jnp.int32),
            pltpu.VMEM((_C,), jnp.int32),
            pltpu.VMEM((_C,), jnp.float32),
            pltpu.VMEM((_C, Dh), jnp.float32),
            pltpu.VMEM_SHARED((NP, Dh), jnp.float32),
            pltpu.SemaphoreType.DMA,
        ],
    )(x0, x1, si, ti, en)
    return jnp.concatenate([out0[:N], out1[:N]], axis=1)


# speedup vs baseline: 3.4444x; 1.1126x over previous
import functools

import jax
import jax.numpy as jnp
from jax import lax
from jax.experimental import pallas as pl
from jax.experimental.pallas import tpu as pltpu
from jax.experimental.pallas import tpu_sc as plsc

_C = 128
_NS = 16
_NBUF = 2
_LANES = 16


def _scale_rows(rv, ev, JB):
    @plsc.parallel_loop(0, _C // _LANES, unroll=2)
    def _scale(k):
        en16 = ev[pl.ds(k * _LANES, _LANES)]
        for l in range(_LANES):
            sc16 = jnp.take_along_axis(
                en16, jnp.full((_LANES,), l, jnp.int32), axis=0,
                mode="promise_in_bounds")
            e = k * _LANES + l
            for j in range(JB):
                sl = pl.ds(j * _LANES, _LANES)
                rv[e, sl] = rv[e, sl] * sc16


def _gc_body(NCH, RP, JB, x0, x1, si, ti, en, out0, out1,
             si4, ti4, en4, rows4, acc, *sems):
    isem = sems[0:_NBUF]
    gsem = sems[_NBUF:2 * _NBUF]
    ssem = sems[2 * _NBUF:3 * _NBUF]
    c = lax.axis_index("c")
    s = lax.axis_index("s")

    r0 = rows4.at[0]

    @plsc.parallel_loop(0, _C, unroll=4)
    def _zrow(i):
        for j in range(JB):
            r0[i, pl.ds(j * _LANES, _LANES)] = jnp.zeros((_LANES,), jnp.float32)

    row0 = pl.multiple_of(s * RP, 8)
    full, rem = divmod(RP, _C)
    for kblk in range(full):
        pltpu.sync_copy(r0, acc.at[pl.ds(row0 + kblk * _C, _C)])
    if rem:
        pltpu.sync_copy(r0.at[pl.ds(0, rem)],
                        acc.at[pl.ds(row0 + full * _C, rem)])
    plsc.subcore_barrier()

    EPT = NCH * _C

    def _super(G, carry):
        ids = []
        for b in range(_NBUF):
            base = pl.multiple_of(s * EPT + (G * _NBUF + b) * _C, _C)
            ids.append([
                pltpu.async_copy(si.at[pl.ds(base, _C)], si4.at[b], isem[b]),
                pltpu.async_copy(ti.at[pl.ds(base, _C)], ti4.at[b], isem[b]),
                pltpu.async_copy(en.at[pl.ds(base, _C)], en4.at[b], isem[b]),
            ])
        for b in range(_NBUF):
            for d in ids[b]:
                d.wait()

            @pl.when(c == 0)
            def _(b=b):
                pltpu.async_copy(x0.at[si4.at[b]], rows4.at[b], gsem[b])

            @pl.when(c == 1)
            def _(b=b):
                pltpu.async_copy(x1.at[si4.at[b]], rows4.at[b], gsem[b])

        sds = []
        for b in range(_NBUF):
            @pl.when(c == 0)
            def _(b=b):
                pltpu.make_async_copy(x0.at[si4.at[b]], rows4.at[b],
                                      gsem[b]).wait()

            @pl.when(c == 1)
            def _(b=b):
                pltpu.make_async_copy(x1.at[si4.at[b]], rows4.at[b],
                                      gsem[b]).wait()

            _scale_rows(rows4.at[b], en4.at[b], JB)
            sds.append(pltpu.async_copy(rows4.at[b], acc.at[ti4.at[b]],
                                        ssem[b], add=True))
        for d in sds:
            d.wait()
        return carry
    lax.fori_loop(0, NCH // _NBUF, _super, 0)

    plsc.subcore_barrier()

    @pl.when(c == 0)
    def _():
        pltpu.sync_copy(acc.at[pl.ds(row0, RP)], out0.at[pl.ds(row0, RP)])

    @pl.when(c == 1)
    def _():
        pltpu.sync_copy(acc.at[pl.ds(row0, RP)], out1.at[pl.ds(row0, RP)])


def kernel(x, eidx, enorm):
    N, D = x.shape
    E = eidx.shape[1]
    Dh = D // 2
    GRAIN = _NS * _C * _NBUF
    EPAD = -(-E // GRAIN) * GRAIN
    NCH = EPAD // (_NS * _C)
    NP = -(-N // (_NS * 8)) * (_NS * 8)
    RP = NP // _NS

    si = jnp.pad(eidx[0].astype(jnp.int32), (0, EPAD - E))
    ti = jnp.pad(eidx[1].astype(jnp.int32), (0, EPAD - E))
    en = jnp.pad(enorm, (0, EPAD - E))
    x0 = x[:, :Dh]
    x1 = x[:, Dh:]

    mesh = plsc.VectorSubcoreMesh(core_axis_name="c", subcore_axis_name="s")
    out0, out1 = pl.kernel(
        functools.partial(_gc_body, NCH, RP, Dh // _LANES),
        out_type=(jax.ShapeDtypeStruct((NP, Dh), jnp.float32),
                  jax.ShapeDtypeStruct((NP, Dh), jnp.float32)),
        mesh=mesh,
        scratch_types=[
            pltpu.VMEM((_NBUF, _C), jnp.int32),
            pltpu.VMEM((_NBUF, _C), jnp.int32),
            pltpu.VMEM((_NBUF, _C), jnp.float32),
            pltpu.VMEM((_NBUF, _C, Dh), jnp.float32),
            pltpu.VMEM_SHARED((NP, Dh), jnp.float32),
        ] + [pltpu.SemaphoreType.DMA] * (3 * _NBUF),
    )(x0, x1, si, ti, en)
    return jnp.concatenate([out0[:N], out1[:N]], axis=1)


# speedup vs baseline: 6.3109x; 1.8322x over previous
import functools

import jax
import jax.numpy as jnp
from jax import lax
from jax.experimental import pallas as pl
from jax.experimental.pallas import tpu as pltpu
from jax.experimental.pallas import tpu_sc as plsc

_C = 112
_NS = 16
_NBUF = 3
_LANES = 16


def _scale_rows(rv, ev, JB):
    @plsc.parallel_loop(0, _C // _LANES, unroll=2)
    def _scale(k):
        en16 = ev[pl.ds(k * _LANES, _LANES)]
        for l in range(_LANES):
            sc16 = jnp.take_along_axis(
                en16, jnp.full((_LANES,), l, jnp.int32), axis=0,
                mode="promise_in_bounds")
            e = k * _LANES + l
            for j in range(JB):
                sl = pl.ds(j * _LANES, _LANES)
                rv[e, sl] = rv[e, sl] * sc16


def _gc_body(NCH, RP, JB, x0, x1, si, ti, en, out0, out1,
             si3, ti3, en3, rows3, acc, *sems):
    isem = sems[0:_NBUF]
    gsem = sems[_NBUF:2 * _NBUF]
    ssem = sems[2 * _NBUF:3 * _NBUF]
    c = lax.axis_index("c")
    s = lax.axis_index("s")
    EPT = NCH * _C

    def fire_idx(g, b):
        base = pl.multiple_of(s * EPT + g * _C, _C)
        pltpu.async_copy(si.at[pl.ds(base, _C)], si3.at[b], isem[b])
        pltpu.async_copy(ti.at[pl.ds(base, _C)], ti3.at[b], isem[b])
        pltpu.async_copy(en.at[pl.ds(base, _C)], en3.at[b], isem[b])

    def wait_idx(g, b):
        base = pl.multiple_of(s * EPT + g * _C, _C)
        pltpu.make_async_copy(si.at[pl.ds(base, _C)], si3.at[b], isem[b]).wait()
        pltpu.make_async_copy(ti.at[pl.ds(base, _C)], ti3.at[b], isem[b]).wait()
        pltpu.make_async_copy(en.at[pl.ds(base, _C)], en3.at[b], isem[b]).wait()

    def fire_gather(b):
        @pl.when(c == 0)
        def _():
            pltpu.async_copy(x0.at[si3.at[b]], rows3.at[b], gsem[b])

        @pl.when(c == 1)
        def _():
            pltpu.async_copy(x1.at[si3.at[b]], rows3.at[b], gsem[b])

    def wait_gather(b):
        @pl.when(c == 0)
        def _():
            pltpu.make_async_copy(x0.at[si3.at[b]], rows3.at[b],
                                  gsem[b]).wait()

        @pl.when(c == 1)
        def _():
            pltpu.make_async_copy(x1.at[si3.at[b]], rows3.at[b],
                                  gsem[b]).wait()

    def fire_scatter(b):
        pltpu.async_copy(rows3.at[b], acc.at[ti3.at[b]], ssem[b], add=True)

    def wait_scatter(b):
        pltpu.make_async_copy(rows3.at[b], acc.at[ti3.at[b]], ssem[b]).wait()

    fire_idx(0, 0)

    r0 = rows3.at[0]

    @plsc.parallel_loop(0, _C, unroll=4)
    def _zrow(i):
        for j in range(JB):
            r0[i, pl.ds(j * _LANES, _LANES)] = jnp.zeros((_LANES,), jnp.float32)

    row0 = pl.multiple_of(s * RP, 8)
    full, rem = divmod(RP, _C)
    for kblk in range(full):
        pltpu.sync_copy(r0, acc.at[pl.ds(row0 + kblk * _C, _C)])
    if rem:
        pltpu.sync_copy(r0.at[pl.ds(0, rem)],
                        acc.at[pl.ds(row0 + full * _C, rem)])
    plsc.subcore_barrier()

    def _super(K, carry):
        for j in range(_NBUF):
            g = _NBUF * K + j
            b, nb, pv = j, (j + 1) % _NBUF, (j + 2) % _NBUF

            @pl.when(g < NCH)
            def _(g=g, b=b):
                wait_idx(g, b)
                fire_gather(b)

            @pl.when(jnp.logical_and(g - 2 >= 0, g - 2 < NCH))
            def _(b=nb):
                wait_scatter(b)

            @pl.when(g + 1 < NCH)
            def _(g=g, b=nb):
                fire_idx(g + 1, b)

            @pl.when(jnp.logical_and(g - 1 >= 0, g - 1 < NCH))
            def _(b=pv):
                wait_gather(b)
                _scale_rows(rows3.at[b], en3.at[b], JB)
                fire_scatter(b)
        return carry
    lax.fori_loop(0, NCH // _NBUF + 1, _super, 0)

    plsc.subcore_barrier()

    @pl.when(c == 0)
    def _():
        pltpu.sync_copy(acc.at[pl.ds(row0, RP)], out0.at[pl.ds(row0, RP)])

    @pl.when(c == 1)
    def _():
        pltpu.sync_copy(acc.at[pl.ds(row0, RP)], out1.at[pl.ds(row0, RP)])


def kernel(x, eidx, enorm):
    N, D = x.shape
    E = eidx.shape[1]
    Dh = D // 2
    GRAIN = _NS * _C * _NBUF
    EPAD = -(-E // GRAIN) * GRAIN
    NCH = EPAD // (_NS * _C)
    NP = -(-N // (_NS * 8)) * (_NS * 8)
    RP = NP // _NS

    si = jnp.pad(eidx[0].astype(jnp.int32), (0, EPAD - E))
    ti = jnp.pad(eidx[1].astype(jnp.int32), (0, EPAD - E))
    en = jnp.pad(enorm, (0, EPAD - E))
    x0 = x[:, :Dh]
    x1 = x[:, Dh:]

    mesh = plsc.VectorSubcoreMesh(core_axis_name="c", subcore_axis_name="s")
    out0, out1 = pl.kernel(
        functools.partial(_gc_body, NCH, RP, Dh // _LANES),
        out_type=(jax.ShapeDtypeStruct((NP, Dh), jnp.float32),
                  jax.ShapeDtypeStruct((NP, Dh), jnp.float32)),
        mesh=mesh,
        scratch_types=[
            pltpu.VMEM((_NBUF, _C), jnp.int32),
            pltpu.VMEM((_NBUF, _C), jnp.int32),
            pltpu.VMEM((_NBUF, _C), jnp.float32),
            pltpu.VMEM((_NBUF, _C, Dh), jnp.float32),
            pltpu.VMEM_SHARED((NP, Dh), jnp.float32),
        ] + [pltpu.SemaphoreType.DMA] * (3 * _NBUF),
    )(x0, x1, si, ti, en)
    return jnp.concatenate([out0[:N], out1[:N]], axis=1)
